# baseline (speedup 1.0000x reference)
import jax
import jax.numpy as jnp
from jax.experimental import pallas as pl


def _loss_kernel(en_ref, de_ref, out_ref):
    j = pl.program_id(1)
    en = en_ref[0]
    de = de_ref[0]
    ed = jnp.sum(en * de, axis=(0, 1))
    ee = jnp.sum(en * en, axis=(0, 1))
    dd = jnp.sum(de * de, axis=(0, 1))

    @pl.when(j == 0)
    def _first():
        out_ref[0, 0] = ed
        out_ref[0, 1] = ee
        out_ref[0, 2] = dd

    @pl.when(j > 0)
    def _rest():
        out_ref[0, 0] += ed
        out_ref[0, 1] += ee
        out_ref[0, 2] += dd


def kernel(encoder_features, decoder_features, global_step):
    B, C, H, W = encoder_features.shape
    en = jnp.transpose(encoder_features, (0, 2, 3, 1))
    de = jnp.transpose(decoder_features, (0, 2, 3, 1))

    nh = 2
    hb = H // nh
    out = pl.pallas_call(
        _loss_kernel,
        grid=(B, nh),
        in_specs=[
            pl.BlockSpec((1, hb, W, C), lambda b, j: (b, j, 0, 0)),
            pl.BlockSpec((1, hb, W, C), lambda b, j: (b, j, 0, 0)),
        ],
        out_specs=pl.BlockSpec((1, 3, C), lambda b, j: (b, 0, 0)),
        out_shape=jax.ShapeDtypeStruct((B, 3, C), jnp.float32),
    )(en, de)

    tot = out.sum(-1)
    dot, na2, nb2 = tot[:, 0], tot[:, 1], tot[:, 2]
    cos = dot / jnp.maximum(jnp.sqrt(na2) * jnp.sqrt(nb2), 1e-8)
    return jnp.mean(1.0 - cos)


# speedup vs baseline: 1.2653x; 1.2653x over previous
import jax
import jax.numpy as jnp
from jax.experimental import pallas as pl
from jax.experimental.pallas import tpu as pltpu

_RING = 3
_HC = 16


def _loss_kernel(en_hbm, de_hbm, out_ref, enb, deb, *sems):
    B = en_hbm.shape[0]
    nch = en_hbm.shape[1] // _HC
    nchunk = B * nch

    def start(c):
        b, h = c // nch, (c % nch) * _HC
        r = c % _RING
        pltpu.make_async_copy(
            en_hbm.at[b, pl.ds(h, _HC)], enb.at[r], sems[r]
        ).start()
        pltpu.make_async_copy(
            de_hbm.at[b, pl.ds(h, _HC)], deb.at[r], sems[_RING + r]
        ).start()

    def wait(c):
        b, h = c // nch, (c % nch) * _HC
        r = c % _RING
        pltpu.make_async_copy(
            en_hbm.at[b, pl.ds(h, _HC)], enb.at[r], sems[r]
        ).wait()
        pltpu.make_async_copy(
            de_hbm.at[b, pl.ds(h, _HC)], deb.at[r], sems[_RING + r]
        ).wait()

    for c in range(_RING):
        start(c)

    acc = None
    for c in range(nchunk):
        wait(c)
        r = c % _RING
        en = enb[r]
        de = deb[r]
        ed = jnp.sum(en * de, axis=(0, 1))
        ee = jnp.sum(en * en, axis=(0, 1))
        dd = jnp.sum(de * de, axis=(0, 1))
        if c % nch == 0:
            acc = (ed, ee, dd)
        else:
            acc = (acc[0] + ed, acc[1] + ee, acc[2] + dd)
        if c % nch == nch - 1:
            b = c // nch
            out_ref[b, 0] = acc[0]
            out_ref[b, 1] = acc[1]
            out_ref[b, 2] = acc[2]
        if c + _RING < nchunk:
            start(c + _RING)


def kernel(encoder_features, decoder_features, global_step):
    B, C, H, W = encoder_features.shape
    en = jnp.transpose(encoder_features, (0, 2, 3, 1))
    de = jnp.transpose(decoder_features, (0, 2, 3, 1))

    out = pl.pallas_call(
        _loss_kernel,
        in_specs=[
            pl.BlockSpec(memory_space=pltpu.MemorySpace.HBM),
            pl.BlockSpec(memory_space=pltpu.MemorySpace.HBM),
        ],
        out_specs=pl.BlockSpec((B, 3, C), lambda: (0, 0, 0)),
        out_shape=jax.ShapeDtypeStruct((B, 3, C), jnp.float32),
        scratch_shapes=(
            [
                pltpu.VMEM((_RING, _HC, W, C), jnp.float32),
                pltpu.VMEM((_RING, _HC, W, C), jnp.float32),
            ]
            + [pltpu.SemaphoreType.DMA] * (2 * _RING)
        ),
    )(en, de)

    tot = out.sum(-1)
    dot, na2, nb2 = tot[:, 0], tot[:, 1], tot[:, 2]
    cos = dot / jnp.maximum(jnp.sqrt(na2) * jnp.sqrt(nb2), 1e-8)
    return jnp.mean(1.0 - cos)


# speedup vs baseline: 1.2696x; 1.0034x over previous
import jax
import jax.numpy as jnp
from jax.experimental import pallas as pl
from jax.experimental.pallas import tpu as pltpu

_RING = 5
_HC = 8


def _loss_kernel(en_hbm, de_hbm, out_ref, enb, deb, *sems):
    B = en_hbm.shape[0]
    nch = en_hbm.shape[1] // _HC
    nchunk = B * nch

    def start(c):
        b, h = c // nch, (c % nch) * _HC
        r = c % _RING
        pltpu.make_async_copy(
            en_hbm.at[b, pl.ds(h, _HC)], enb.at[r], sems[r]
        ).start()
        pltpu.make_async_copy(
            de_hbm.at[b, pl.ds(h, _HC)], deb.at[r], sems[_RING + r]
        ).start()

    def wait(c):
        b, h = c // nch, (c % nch) * _HC
        r = c % _RING
        pltpu.make_async_copy(
            en_hbm.at[b, pl.ds(h, _HC)], enb.at[r], sems[r]
        ).wait()
        pltpu.make_async_copy(
            de_hbm.at[b, pl.ds(h, _HC)], deb.at[r], sems[_RING + r]
        ).wait()

    for c in range(_RING):
        start(c)

    acc = None
    for c in range(nchunk):
        wait(c)
        r = c % _RING
        en = enb[r]
        de = deb[r]
        ed = jnp.sum(en * de, axis=(0, 1))
        ee = jnp.sum(en * en, axis=(0, 1))
        dd = jnp.sum(de * de, axis=(0, 1))
        if c % nch == 0:
            acc = (ed, ee, dd)
        else:
            acc = (acc[0] + ed, acc[1] + ee, acc[2] + dd)
        if c % nch == nch - 1:
            b = c // nch
            out_ref[b, 0] = acc[0]
            out_ref[b, 1] = acc[1]
            out_ref[b, 2] = acc[2]
        if c + _RING < nchunk:
            start(c + _RING)


def kernel(encoder_features, decoder_features, global_step):
    B, C, H, W = encoder_features.shape
    en = jnp.transpose(encoder_features, (0, 2, 3, 1))
    de = jnp.transpose(decoder_features, (0, 2, 3, 1))

    out = pl.pallas_call(
        _loss_kernel,
        in_specs=[
            pl.BlockSpec(memory_space=pltpu.MemorySpace.HBM),
            pl.BlockSpec(memory_space=pltpu.MemorySpace.HBM),
        ],
        out_specs=pl.BlockSpec((B, 3, C), lambda: (0, 0, 0)),
        out_shape=jax.ShapeDtypeStruct((B, 3, C), jnp.float32),
        scratch_shapes=(
            [
                pltpu.VMEM((_RING, _HC, W, C), jnp.float32),
                pltpu.VMEM((_RING, _HC, W, C), jnp.float32),
            ]
            + [pltpu.SemaphoreType.DMA] * (2 * _RING)
        ),
    )(en, de)

    tot = out.sum(-1)
    dot, na2, nb2 = tot[:, 0], tot[:, 1], tot[:, 2]
    cos = dot / jnp.maximum(jnp.sqrt(na2) * jnp.sqrt(nb2), 1e-8)
    return jnp.mean(1.0 - cos)


# speedup vs baseline: 1.4114x; 1.1117x over previous
import jax
import jax.numpy as jnp
from jax.experimental import pallas as pl
from jax.experimental.pallas import tpu as pltpu

_RING = 5
_HC = 8


def _loss_kernel(en_hbm, de_hbm, out_ref, enb, deb, *sems):
    B = en_hbm.shape[0]
    nch = en_hbm.shape[1] // _HC
    nchunk = B * nch

    def start(c):
        b, h = c // nch, (c % nch) * _HC
        r = c % _RING
        pltpu.make_async_copy(
            en_hbm.at[b, pl.ds(h, _HC)], enb.at[r], sems[r]
        ).start()
        pltpu.make_async_copy(
            de_hbm.at[b, pl.ds(h, _HC)], deb.at[r], sems[_RING + r]
        ).start()

    def wait(c):
        b, h = c // nch, (c % nch) * _HC
        r = c % _RING
        pltpu.make_async_copy(
            en_hbm.at[b, pl.ds(h, _HC)], enb.at[r], sems[r]
        ).wait()
        pltpu.make_async_copy(
            de_hbm.at[b, pl.ds(h, _HC)], deb.at[r], sems[_RING + r]
        ).wait()

    for c in range(_RING):
        start(c)

    acc = None
    loss_sum = jnp.float32(0.0)
    for c in range(nchunk):
        wait(c)
        r = c % _RING
        en = enb[r]
        de = deb[r]
        ed = jnp.sum(en * de, axis=(0, 1))
        ee = jnp.sum(en * en, axis=(0, 1))
        dd = jnp.sum(de * de, axis=(0, 1))
        if c % nch == 0:
            acc = (ed, ee, dd)
        else:
            acc = (acc[0] + ed, acc[1] + ee, acc[2] + dd)
        if c % nch == nch - 1:
            dot = jnp.sum(acc[0])
            na2 = jnp.sum(acc[1])
            nb2 = jnp.sum(acc[2])
            cos = dot / jnp.maximum(jnp.sqrt(na2) * jnp.sqrt(nb2), 1e-8)
            loss_sum = loss_sum + (1.0 - cos)
        if c + _RING < nchunk:
            start(c + _RING)

    out_ref[0, 0] = loss_sum / B


def kernel(encoder_features, decoder_features, global_step):
    B, C, H, W = encoder_features.shape
    en = jnp.transpose(encoder_features, (0, 2, 3, 1))
    de = jnp.transpose(decoder_features, (0, 2, 3, 1))

    out = pl.pallas_call(
        _loss_kernel,
        in_specs=[
            pl.BlockSpec(memory_space=pltpu.MemorySpace.HBM),
            pl.BlockSpec(memory_space=pltpu.MemorySpace.HBM),
        ],
        out_specs=pl.BlockSpec(memory_space=pltpu.SMEM),
        out_shape=jax.ShapeDtypeStruct((1, 1), jnp.float32),
        scratch_shapes=(
            [
                pltpu.VMEM((_RING, _HC, W, C), jnp.float32),
                pltpu.VMEM((_RING, _HC, W, C), jnp.float32),
            ]
            + [pltpu.SemaphoreType.DMA] * (2 * _RING)
        ),
    )(en, de)
    return out[0, 0].reshape(())
